# baseline (speedup 1.0000x reference)
import functools

import jax
import jax.numpy as jnp
from jax import lax
from jax.experimental import pallas as pl
from jax.experimental.pallas import tpu as pltpu
from jax.experimental.pallas import tpu_sc as plsc

N_NODES = 10000
N_EDGES = 160000
VOCAB = 512
EMB_DIM = 256
HIDDEN = 512
OUT_DIM = 256

NPAD = 10240
QS = NPAD // 16
RROWS = 160
RFLAT = RROWS * VOCAB
NRANGE = NPAD // RROWS
ECH = 1024
EPT = N_EDGES // 16
L = 16


def _rsqrt16(d):
    i = plsc.bitcast(d, jnp.int32)
    i = jnp.int32(0x5F3759DF) - lax.shift_right_logical(i, 1)
    y = plsc.bitcast(i, jnp.float32)
    for _ in range(3):
        y = y * (1.5 - 0.5 * d * y * y)
    return y


def _sc_body(ids_hbm, src_hbm, dst_hbm, g_hbm, c_hbm, fl_hbm, w_hbm,
             ids_v, dinv_v, hist_v, tmp_v, g_v, es_v, ed_v, fl_v, w_v,
             part_sh, dinv_sh):
    cid = lax.axis_index("c")
    sid = lax.axis_index("s")
    wid = cid * 16 + sid
    cbase = cid * N_EDGES
    ones = jnp.full((L,), 1.0, jnp.float32)
    zeros = jnp.zeros((L,), jnp.float32)

    pltpu.sync_copy(ids_hbm, ids_v)

    def zero_hist(j, _):
        hist_v[pl.ds(j * L, L)] = zeros
        return 0

    ebase = sid * EPT
    slice_chunks = [(i * ECH, ECH) for i in range(EPT // ECH)]
    if EPT % ECH:
        slice_chunks.append(((EPT // ECH) * ECH, EPT % ECH))

    lax.fori_loop(0, NPAD // L, zero_hist, 0)
    for off, n in slice_chunks:
        pltpu.sync_copy(dst_hbm.at[pl.ds(ebase + off, n)], ed_v.at[pl.ds(0, n)])

        def deg_body(i, _):
            d = ed_v[pl.ds(i * L, L)]
            plsc.addupdate_scatter(hist_v, [d], ones)
            return 0

        lax.fori_loop(0, n // L, deg_body, 0)

    go = sid * QS

    def spmem_reduce(acc_ref):
        pltpu.sync_copy(hist_v, part_sh.at[sid])
        plsc.subcore_barrier()

        def zero_acc(j, _):
            acc_ref[pl.ds(go + j * L, L)] = zeros
            return 0

        lax.fori_loop(0, QS // L, zero_acc, 0)
        for k in range(16):
            pltpu.sync_copy(part_sh.at[k, pl.ds(go, QS)], tmp_v)

            def add_k(j, _):
                acc_ref[pl.ds(go + j * L, L)] += tmp_v[pl.ds(j * L, L)]
                return 0

            lax.fori_loop(0, QS // L, add_k, 0)
        plsc.subcore_barrier()

    spmem_reduce(dinv_v)

    def newton(j, _):
        deg = dinv_v[pl.ds(go + j * L, L)] + 1.0
        dinv_v[pl.ds(go + j * L, L)] = _rsqrt16(deg)
        return 0

    lax.fori_loop(0, QS // L, newton, 0)
    pltpu.sync_copy(dinv_v.at[pl.ds(go, QS)], dinv_sh.at[pl.ds(go, QS)])
    plsc.subcore_barrier()
    pltpu.sync_copy(dinv_sh, dinv_v)

    lax.fori_loop(0, NPAD // L, zero_hist, 0)
    for off, n in slice_chunks:
        pltpu.sync_copy(src_hbm.at[pl.ds(ebase + off, n)], es_v.at[pl.ds(0, n)])
        pltpu.sync_copy(dst_hbm.at[pl.ds(ebase + off, n)], ed_v.at[pl.ds(0, n)])

        def s_body(i, _):
            s = es_v[pl.ds(i * L, L)]
            d = ed_v[pl.ds(i * L, L)]
            dd = plsc.load_gather(dinv_v, [d])
            plsc.addupdate_scatter(hist_v, [s], dd)
            vid = plsc.load_gather(ids_v, [s])
            w = plsc.load_gather(dinv_v, [s]) * dd
            fl_v[pl.ds(i * L, L)] = d * VOCAB + vid
            w_v[pl.ds(i * L, L)] = w
            return 0

        lax.fori_loop(0, n // L, s_body, 0)
        pltpu.sync_copy(fl_v.at[pl.ds(0, n)], fl_hbm.at[pl.ds(cbase + ebase + off, n)])
        pltpu.sync_copy(w_v.at[pl.ds(0, n)], w_hbm.at[pl.ds(cbase + ebase + off, n)])

    spmem_reduce(hist_v)

    @pl.when(cid == 0)
    def _():
        def c_body(j, _):
            idx = lax.iota(jnp.int32, L) + (go + j * L)
            dv = dinv_v[pl.ds(go + j * L, L)]
            sv = hist_v[pl.ds(go + j * L, L)]
            tmp_v[pl.ds(j * L, L)] = jnp.where(idx < N_NODES, dv * (dv + sv), 0.0)
            return 0

        lax.fori_loop(0, QS // L, c_body, 0)
        pltpu.sync_copy(tmp_v, c_hbm.at[pl.ds(go, QS)])

    plsc.subcore_barrier()

    full_chunks = [(i * ECH, ECH) for i in range(N_EDGES // ECH)]
    tail = N_EDGES % ECH
    for rr in range(2):
        rng = wid * 2 + rr
        base = rng * RFLAT

        def zero_g(j, _):
            g_v[pl.ds(j * L, L)] = zeros
            return 0

        lax.fori_loop(0, RFLAT // L, zero_g, 0)

        def edge_body(i, _):
            fg = fl_v[pl.ds(i * L, L)]
            wv = w_v[pl.ds(i * L, L)]
            local = fg - base
            msk = (local >= 0) & (local < RFLAT)
            lc = jnp.where(msk, local, 0)
            plsc.addupdate_scatter(g_v, [lc], wv, mask=msk)
            return 0

        def chunk(ch, _):
            off = ch * ECH
            pltpu.sync_copy(fl_hbm.at[pl.ds(cbase + off, ECH)], fl_v)
            pltpu.sync_copy(w_hbm.at[pl.ds(cbase + off, ECH)], w_v)
            lax.fori_loop(0, ECH // L, edge_body, 0)
            return 0

        lax.fori_loop(0, N_EDGES // ECH, chunk, 0)
        if tail:
            toff = (N_EDGES // ECH) * ECH
            pltpu.sync_copy(fl_hbm.at[pl.ds(cbase + toff, tail)], fl_v.at[pl.ds(0, tail)])
            pltpu.sync_copy(w_hbm.at[pl.ds(cbase + toff, tail)], w_v.at[pl.ds(0, tail)])
            lax.fori_loop(0, tail // L, edge_body, 0)

        rbase = rng * RROWS

        def self_body(j, _):
            li = lax.iota(jnp.int32, L) + j * L
            node = li + rbase
            m = node < N_NODES
            nc = jnp.where(m, node, 0)
            vidn = plsc.load_gather(ids_v, [nc])
            dv = plsc.load_gather(dinv_v, [nc])
            plsc.addupdate_scatter(g_v, [li * VOCAB + vidn], dv * dv, mask=m)
            return 0

        lax.fori_loop(0, RROWS // L, self_body, 0)
        pltpu.sync_copy(g_v, g_hbm.at[pl.ds(base, RFLAT)])


def _make_sc_call(interpret=False):
    mesh = plsc.VectorSubcoreMesh(core_axis_name="c", subcore_axis_name="s",
                                  num_cores=2, num_subcores=16)
    return pl.kernel(
        _sc_body,
        out_type=[
            jax.ShapeDtypeStruct((NPAD * VOCAB,), jnp.float32),
            jax.ShapeDtypeStruct((NPAD,), jnp.float32),
            jax.ShapeDtypeStruct((2 * N_EDGES,), jnp.int32),
            jax.ShapeDtypeStruct((2 * N_EDGES,), jnp.float32),
        ],
        mesh=mesh,
        scratch_types=[
            pltpu.VMEM((N_NODES,), jnp.int32),
            pltpu.VMEM((NPAD,), jnp.float32),
            pltpu.VMEM((NPAD,), jnp.float32),
            pltpu.VMEM((QS,), jnp.float32),
            pltpu.VMEM((RFLAT,), jnp.float32),
            pltpu.VMEM((ECH,), jnp.int32),
            pltpu.VMEM((ECH,), jnp.int32),
            pltpu.VMEM((ECH,), jnp.int32),
            pltpu.VMEM((ECH,), jnp.float32),
            pltpu.VMEM_SHARED((16, NPAD), jnp.float32),
            pltpu.VMEM_SHARED((NPAD,), jnp.float32),
        ],
        compiler_params=pltpu.CompilerParams(needs_layout_passes=False),
        interpret=interpret,
    )


def _tc_body(g_ref, c_ref, e_ref, w1_ref, b1_ref, w2_ref, b2_ref, out_ref,
             t_s, p_s):
    i = pl.program_id(0)

    @pl.when(i == 0)
    def _():
        t_s[...] = jnp.dot(e_ref[...], w1_ref[...],
                           preferred_element_type=jnp.float32)
        p_s[...] = jnp.zeros((1, HIDDEN), jnp.float32)

    z = jnp.dot(g_ref[...], t_s[...], preferred_element_type=jnp.float32)
    h = jnp.maximum(z + b1_ref[...], 0.0)
    p_s[...] += jnp.sum(h * c_ref[0], axis=0, keepdims=True)

    @pl.when(i == NRANGE - 1)
    def _():
        out_ref[...] = jnp.dot(p_s[...] * (1.0 / N_NODES), w2_ref[...],
                               preferred_element_type=jnp.float32) + b2_ref[...]


def _make_tc_call(interpret=False):
    return pl.pallas_call(
        _tc_body,
        grid=(NRANGE,),
        in_specs=[
            pl.BlockSpec((RROWS, VOCAB), lambda i: (i, 0)),
            pl.BlockSpec((1, RROWS, 1), lambda i: (i, 0, 0)),
            pl.BlockSpec((VOCAB, EMB_DIM), lambda i: (0, 0)),
            pl.BlockSpec((EMB_DIM, HIDDEN), lambda i: (0, 0)),
            pl.BlockSpec((1, HIDDEN), lambda i: (0, 0)),
            pl.BlockSpec((HIDDEN, OUT_DIM), lambda i: (0, 0)),
            pl.BlockSpec((1, OUT_DIM), lambda i: (0, 0)),
        ],
        out_specs=pl.BlockSpec((1, OUT_DIM), lambda i: (0, 0)),
        out_shape=jax.ShapeDtypeStruct((1, OUT_DIM), jnp.float32),
        scratch_shapes=[
            pltpu.VMEM((VOCAB, HIDDEN), jnp.float32),
            pltpu.VMEM((1, HIDDEN), jnp.float32),
        ],
        interpret=interpret,
    )


@jax.jit
def kernel(x_node_ids, edge_index, emb_table, W1, b1, W2, b2):
    ids = x_node_ids.astype(jnp.int32)
    src = edge_index[0].astype(jnp.int32)
    dst = edge_index[1].astype(jnp.int32)
    g_flat, c, _, _ = _make_sc_call()(ids, src, dst)
    G = g_flat.reshape(NPAD, VOCAB)
    c3 = c.reshape(NRANGE, RROWS, 1)
    return _make_tc_call()(
        G, c3, emb_table, W1, b1.reshape(1, HIDDEN), W2,
        b2.reshape(1, OUT_DIM))


# speedup vs baseline: 1.8629x; 1.8629x over previous
import functools

import jax
import jax.numpy as jnp
from jax import lax
from jax.experimental import pallas as pl
from jax.experimental.pallas import tpu as pltpu
from jax.experimental.pallas import tpu_sc as plsc

N_NODES = 10000
N_EDGES = 160000
VOCAB = 512
EMB_DIM = 256
HIDDEN = 512
OUT_DIM = 256

NPAD = 10240
QS = NPAD // 16
RROWS = 160
RFLAT = RROWS * VOCAB
NRANGE = NPAD // RROWS
ECH = 1024
EPT = N_EDGES // 16
L = 16
HOFF = NPAD
SPAD = 172032
RING = 4096
NCH = SPAD // RING


def _rsqrt16(d):
    i = plsc.bitcast(d, jnp.int32)
    i = jnp.int32(0x5F3759DF) - lax.shift_right_logical(i, 1)
    y = plsc.bitcast(i, jnp.float32)
    for _ in range(3):
        y = y * (1.5 - 0.5 * d * y * y)
    return y


def _sc_body(ids_hbm, src_hbm, dst_hbm, g_hbm, c_hbm, fl_hbm, w_hbm,
             arena_i, arena_f, tmp_v, g_v, es_v, ed_v, flst_v, wst_v,
             part_sh, dinv_sh, sem_f0, sem_w0, sem_f1, sem_w1):
    cid = lax.axis_index("c")
    sid = lax.axis_index("s")
    wid = cid * 16 + sid
    sbase = cid * SPAD
    ones = jnp.full((L,), 1.0, jnp.float32)
    zeros = jnp.zeros((L,), jnp.float32)

    pltpu.sync_copy(ids_hbm, arena_i.at[pl.ds(0, N_NODES)])

    def zero_hist(j, _):
        arena_f[pl.ds(HOFF + j * L, L)] = zeros
        return 0

    ebase = sid * EPT
    slice_chunks = [(i * ECH, ECH) for i in range(EPT // ECH)]
    if EPT % ECH:
        slice_chunks.append(((EPT // ECH) * ECH, EPT % ECH))

    lax.fori_loop(0, NPAD // L, zero_hist, 0)
    for off, n in slice_chunks:
        pltpu.sync_copy(dst_hbm.at[pl.ds(ebase + off, n)], ed_v.at[pl.ds(0, n)])

        def deg_body(i, _):
            d = ed_v[pl.ds(i * L, L)]
            plsc.addupdate_scatter(arena_f, [d + HOFF], ones)
            return 0

        lax.fori_loop(0, n // L, deg_body, 0)

    go = sid * QS

    def spmem_reduce(boff):
        pltpu.sync_copy(arena_f.at[pl.ds(HOFF, NPAD)], part_sh.at[sid])
        plsc.subcore_barrier()

        def zero_acc(j, _):
            arena_f[pl.ds(boff + go + j * L, L)] = zeros
            return 0

        lax.fori_loop(0, QS // L, zero_acc, 0)
        for k in range(16):
            pltpu.sync_copy(part_sh.at[k, pl.ds(go, QS)], tmp_v)

            def add_k(j, _):
                arena_f[pl.ds(boff + go + j * L, L)] += tmp_v[pl.ds(j * L, L)]
                return 0

            lax.fori_loop(0, QS // L, add_k, 0)
        plsc.subcore_barrier()

    spmem_reduce(0)

    def newton(j, _):
        deg = arena_f[pl.ds(go + j * L, L)] + 1.0
        arena_f[pl.ds(go + j * L, L)] = _rsqrt16(deg)
        return 0

    lax.fori_loop(0, QS // L, newton, 0)
    pltpu.sync_copy(arena_f.at[pl.ds(go, QS)], dinv_sh.at[pl.ds(go, QS)])
    plsc.subcore_barrier()
    pltpu.sync_copy(dinv_sh, arena_f.at[pl.ds(0, NPAD)])

    lax.fori_loop(0, NPAD // L, zero_hist, 0)
    for off, n in slice_chunks:
        pltpu.sync_copy(src_hbm.at[pl.ds(ebase + off, n)], es_v.at[pl.ds(0, n)])
        pltpu.sync_copy(dst_hbm.at[pl.ds(ebase + off, n)], ed_v.at[pl.ds(0, n)])

        def s_body(i, _):
            s = es_v[pl.ds(i * L, L)]
            d = ed_v[pl.ds(i * L, L)]
            dd = plsc.load_gather(arena_f, [d])
            plsc.addupdate_scatter(arena_f, [s + HOFF], dd)
            vid = plsc.load_gather(arena_i, [s])
            w = plsc.load_gather(arena_f, [s]) * dd
            flst_v[pl.ds(i * L, L)] = d * VOCAB + vid
            wst_v[pl.ds(i * L, L)] = w
            return 0

        lax.fori_loop(0, n // L, s_body, 0)
        pltpu.sync_copy(flst_v.at[pl.ds(0, n)], fl_hbm.at[pl.ds(sbase + ebase + off, n)])
        pltpu.sync_copy(wst_v.at[pl.ds(0, n)], w_hbm.at[pl.ds(sbase + ebase + off, n)])

    spmem_reduce(HOFF)

    @pl.when(cid == 0)
    def _():
        def c_body(j, _):
            idx = lax.iota(jnp.int32, L) + (go + j * L)
            dv = arena_f[pl.ds(go + j * L, L)]
            sv = arena_f[pl.ds(HOFF + go + j * L, L)]
            tmp_v[pl.ds(j * L, L)] = jnp.where(idx < N_NODES, dv * (dv + sv), 0.0)
            return 0

        lax.fori_loop(0, QS // L, c_body, 0)
        pltpu.sync_copy(tmp_v, c_hbm.at[pl.ds(go, QS)])

    def self_body(j, _):
        node = lax.iota(jnp.int32, L) + (go + j * L)
        m = node < N_NODES
        nc = jnp.where(m, node, 0)
        vidn = plsc.load_gather(arena_i, [nc])
        dv = plsc.load_gather(arena_f, [nc])
        flst_v[pl.ds(j * L, L)] = jnp.where(m, nc * VOCAB + vidn, 0)
        wst_v[pl.ds(j * L, L)] = jnp.where(m, dv * dv, 0.0)
        return 0

    lax.fori_loop(0, QS // L, self_body, 0)
    pltpu.sync_copy(flst_v.at[pl.ds(0, QS)],
                    fl_hbm.at[pl.ds(sbase + N_EDGES + go, QS)])
    pltpu.sync_copy(wst_v.at[pl.ds(0, QS)],
                    w_hbm.at[pl.ds(sbase + N_EDGES + go, QS)])

    @pl.when(sid == 15)
    def _():
        def zero_st(j, _):
            flst_v[pl.ds(j * L, L)] = jnp.zeros((L,), jnp.int32)
            wst_v[pl.ds(j * L, L)] = zeros
            return 0

        lax.fori_loop(0, ECH // L, zero_st, 0)
        pbase = sbase + N_EDGES + NPAD
        npadlen = SPAD - N_EDGES - NPAD
        done = 0
        while done < npadlen:
            n = min(ECH, npadlen - done)
            pltpu.sync_copy(flst_v.at[pl.ds(0, n)], fl_hbm.at[pl.ds(pbase + done, n)])
            pltpu.sync_copy(wst_v.at[pl.ds(0, n)], w_hbm.at[pl.ds(pbase + done, n)])
            done += n

    plsc.subcore_barrier()

    rings = [(0, sem_f0, sem_w0), (RING, sem_f1, sem_w1)]
    for rr in range(2):
        rng = wid * 2 + rr
        base = rng * RFLAT

        def zero_g(j, _):
            g_v[pl.ds(j * L, L)] = zeros
            return 0

        lax.fori_loop(0, RFLAT // L, zero_g, 0)

        for b, (fb, sf, sw) in enumerate(rings):
            pltpu.async_copy(fl_hbm.at[pl.ds(sbase + b * RING, RING)],
                             arena_i.at[pl.ds(fb, RING)], sf)
            pltpu.async_copy(w_hbm.at[pl.ds(sbase + b * RING, RING)],
                             arena_f.at[pl.ds(fb, RING)], sw)

        def superstep(sc2, _):
            for b, (fb, sf, sw) in enumerate(rings):
                ch = sc2 * 2 + b
                pltpu.make_async_copy(fl_hbm.at[pl.ds(sbase, RING)],
                                      arena_i.at[pl.ds(fb, RING)], sf).wait()
                pltpu.make_async_copy(w_hbm.at[pl.ds(sbase, RING)],
                                      arena_f.at[pl.ds(fb, RING)], sw).wait()

                def edge_body(i, _):
                    fg = arena_i[pl.ds(fb + i * L, L)]
                    wv = arena_f[pl.ds(fb + i * L, L)]
                    local = fg - base
                    msk = (local >= 0) & (local < RFLAT)
                    lc = jnp.where(msk, local, 0)
                    plsc.addupdate_scatter(g_v, [lc], wv, mask=msk)
                    return 0

                lax.fori_loop(0, RING // L, edge_body, 0)

                @pl.when(ch + 2 < NCH)
                def _():
                    off = sbase + (ch + 2) * RING
                    pltpu.async_copy(fl_hbm.at[pl.ds(off, RING)],
                                     arena_i.at[pl.ds(fb, RING)], sf)
                    pltpu.async_copy(w_hbm.at[pl.ds(off, RING)],
                                     arena_f.at[pl.ds(fb, RING)], sw)
            return 0

        lax.fori_loop(0, NCH // 2, superstep, 0)
        pltpu.sync_copy(g_v, g_hbm.at[pl.ds(base, RFLAT)])


def _make_sc_call(interpret=False):
    mesh = plsc.VectorSubcoreMesh(core_axis_name="c", subcore_axis_name="s",
                                  num_cores=2, num_subcores=16)
    return pl.kernel(
        _sc_body,
        out_type=[
            jax.ShapeDtypeStruct((NPAD * VOCAB,), jnp.float32),
            jax.ShapeDtypeStruct((NPAD,), jnp.float32),
            jax.ShapeDtypeStruct((2 * SPAD,), jnp.int32),
            jax.ShapeDtypeStruct((2 * SPAD,), jnp.float32),
        ],
        mesh=mesh,
        scratch_types=[
            pltpu.VMEM((N_NODES + 240,), jnp.int32),
            pltpu.VMEM((2 * NPAD,), jnp.float32),
            pltpu.VMEM((QS,), jnp.float32),
            pltpu.VMEM((RFLAT,), jnp.float32),
            pltpu.VMEM((ECH,), jnp.int32),
            pltpu.VMEM((ECH,), jnp.int32),
            pltpu.VMEM((ECH,), jnp.int32),
            pltpu.VMEM((ECH,), jnp.float32),
            pltpu.VMEM_SHARED((16, NPAD), jnp.float32),
            pltpu.VMEM_SHARED((NPAD,), jnp.float32),
            pltpu.SemaphoreType.DMA,
            pltpu.SemaphoreType.DMA,
            pltpu.SemaphoreType.DMA,
            pltpu.SemaphoreType.DMA,
        ],
        compiler_params=pltpu.CompilerParams(needs_layout_passes=False),
        interpret=interpret,
    )


def _tc_body(g_ref, c_ref, e_ref, w1_ref, b1_ref, w2_ref, b2_ref, out_ref,
             t_s, p_s):
    i = pl.program_id(0)

    @pl.when(i == 0)
    def _():
        t_s[...] = jnp.dot(e_ref[...], w1_ref[...],
                           preferred_element_type=jnp.float32)
        p_s[...] = jnp.zeros((1, HIDDEN), jnp.float32)

    z = jnp.dot(g_ref[...], t_s[...], preferred_element_type=jnp.float32)
    h = jnp.maximum(z + b1_ref[...], 0.0)
    p_s[...] += jnp.sum(h * c_ref[0], axis=0, keepdims=True)

    @pl.when(i == NRANGE - 1)
    def _():
        out_ref[...] = jnp.dot(p_s[...] * (1.0 / N_NODES), w2_ref[...],
                               preferred_element_type=jnp.float32) + b2_ref[...]


def _make_tc_call(interpret=False):
    return pl.pallas_call(
        _tc_body,
        grid=(NRANGE,),
        in_specs=[
            pl.BlockSpec((RROWS, VOCAB), lambda i: (i, 0)),
            pl.BlockSpec((1, RROWS, 1), lambda i: (i, 0, 0)),
            pl.BlockSpec((VOCAB, EMB_DIM), lambda i: (0, 0)),
            pl.BlockSpec((EMB_DIM, HIDDEN), lambda i: (0, 0)),
            pl.BlockSpec((1, HIDDEN), lambda i: (0, 0)),
            pl.BlockSpec((HIDDEN, OUT_DIM), lambda i: (0, 0)),
            pl.BlockSpec((1, OUT_DIM), lambda i: (0, 0)),
        ],
        out_specs=pl.BlockSpec((1, OUT_DIM), lambda i: (0, 0)),
        out_shape=jax.ShapeDtypeStruct((1, OUT_DIM), jnp.float32),
        scratch_shapes=[
            pltpu.VMEM((VOCAB, HIDDEN), jnp.float32),
            pltpu.VMEM((1, HIDDEN), jnp.float32),
        ],
        interpret=interpret,
    )


@jax.jit
def kernel(x_node_ids, edge_index, emb_table, W1, b1, W2, b2):
    ids = x_node_ids.astype(jnp.int32)
    src = edge_index[0].astype(jnp.int32)
    dst = edge_index[1].astype(jnp.int32)
    g_flat, c, _, _ = _make_sc_call()(ids, src, dst)
    G = g_flat.reshape(NPAD, VOCAB)
    c3 = c.reshape(NRANGE, RROWS, 1)
    return _make_tc_call()(
        G, c3, emb_table, W1, b1.reshape(1, HIDDEN), W2,
        b2.reshape(1, OUT_DIM))
